# baseline (speedup 1.0000x reference)
import functools

import jax
import jax.numpy as jnp
from jax import lax
from jax.experimental import pallas as pl
from jax.experimental.pallas import tpu as pltpu
from jax.experimental.pallas import tpu_sc as plsc

N = 50000
E = 800000
D = 100
DP = 128
C = 32
NCHUNK = DP // C
BLK = 128
NBLK = E // BLK
N_PAD = 50048
ROWS_PER_TILE = N_PAD // 16


def _edge_mm_body(ea_ref, w_ref, b_ref, out_ref):
    out_ref[...] = jnp.dot(ea_ref[...], w_ref[...],
                           preferred_element_type=jnp.float32,
                           precision=lax.Precision.HIGHEST) + b_ref[...]


def _edge_mm(edge_attr, w_pad, b_pad):
    bk = 2000
    return pl.pallas_call(
        _edge_mm_body,
        grid=(E // bk,),
        in_specs=[
            pl.BlockSpec((bk, D), lambda i: (i, 0)),
            pl.BlockSpec((D, DP), lambda i: (0, 0)),
            pl.BlockSpec((1, DP), lambda i: (0, 0)),
        ],
        out_specs=pl.BlockSpec((bk, DP), lambda i: (i, 0)),
        out_shape=jax.ShapeDtypeStruct((E, DP), jnp.float32),
    )(edge_attr, w_pad, b_pad)


def _node_mm_body(x_ref, wi_ref, wj_ref, oi_ref, oj_ref):
    yi = jnp.dot(x_ref[...], wi_ref[...],
                 preferred_element_type=jnp.float32,
                 precision=lax.Precision.HIGHEST)
    yj = jnp.dot(x_ref[...], wj_ref[...],
                 preferred_element_type=jnp.float32,
                 precision=lax.Precision.HIGHEST)
    for c in range(NCHUNK):
        oi_ref[c] = yi[:, c * C:(c + 1) * C]
        oj_ref[c] = yj[:, c * C:(c + 1) * C]


def _node_mm(x, wi_pad, wj_pad):
    bn = 2000
    return pl.pallas_call(
        _node_mm_body,
        grid=(N // bn,),
        in_specs=[
            pl.BlockSpec((bn, D), lambda i: (i, 0)),
            pl.BlockSpec((D, DP), lambda i: (0, 0)),
            pl.BlockSpec((D, DP), lambda i: (0, 0)),
        ],
        out_specs=[
            pl.BlockSpec((NCHUNK, bn, C), lambda i: (0, i, 0)),
            pl.BlockSpec((NCHUNK, bn, C), lambda i: (0, i, 0)),
        ],
        out_shape=[
            jax.ShapeDtypeStruct((NCHUNK, N, C), jnp.float32),
            jax.ShapeDtypeStruct((NCHUNK, N, C), jnp.float32),
        ],
    )(x, wi_pad, wj_pad)


def _sc_body(dst_hbm, src_hbm, eaw_hbm, xiw_hbm, xjw_hbm, zeros_hbm,
             out_hbm, shared,
             d0, d1, s0, s1, ad0, ad1, as0, as1, e0, e1, a0, a1, b0, b1,
             se0, se1, sa0, sa1, sb0, sb1, ss0, ss1):
    core = lax.axis_index("c")
    sid = lax.axis_index("s")
    slots = ((d0, s0, ad0, as0, e0, a0, b0, se0, sa0, sb0, ss0),
             (d1, s1, ad1, as1, e1, a1, b1, se1, sa1, sb1, ss1))

    def zero_table():
        pltpu.sync_copy(zeros_hbm,
                        shared.at[pl.ds(sid * ROWS_PER_TILE, ROWS_PER_TILE)])

    def run_round(chunk):
        ccol = pl.ds(chunk * C, C)
        cn = chunk * N

        def issue(j, s, wait_scatter):
            db, srb, adb, asb, eb, ab, bb, se, sa, sb, ss = slots[s]
            if wait_scatter:
                pltpu.make_async_copy(eb, shared.at[db.at[0]], ss).wait()
            blk = sid + 16 * j
            pltpu.sync_copy(dst_hbm.at[pl.ds(blk, 1)], db)
            pltpu.sync_copy(src_hbm.at[pl.ds(blk, 1)], srb)
            for v in range(8):
                sl = pl.ds(v * 16, 16)
                adb[0, sl] = db[0, sl] + cn
                asb[0, sl] = srb[0, sl] + cn
            pltpu.async_copy(eaw_hbm.at[pl.ds(blk * BLK, BLK), ccol], eb, se)
            pltpu.async_copy(xiw_hbm.at[adb.at[0]], ab, sa)
            pltpu.async_copy(xjw_hbm.at[asb.at[0]], bb, sb)

        def consume(s):
            db, srb, adb, asb, eb, ab, bb, se, sa, sb, ss = slots[s]
            pltpu.make_async_copy(eaw_hbm.at[pl.ds(0, BLK), ccol], eb,
                                  se).wait()
            pltpu.make_async_copy(xiw_hbm.at[adb.at[0]], ab, sa).wait()
            pltpu.make_async_copy(xjw_hbm.at[asb.at[0]], bb, sb).wait()

            def relu_row(r, acc):
                for h in range(2):
                    sl = pl.ds(h * 16, 16)
                    m = eb[r, sl] + ab[r, sl] + bb[r, sl]
                    eb[r, sl] = jnp.maximum(m, 0.0)
                return acc

            lax.fori_loop(0, BLK, relu_row, 0)
            pltpu.async_copy(eb, shared.at[db.at[0]], ss, add=True)

        nblk = (NBLK // 16) + jnp.where(sid < (NBLK % 16), 1, 0)
        issue(0, 0, wait_scatter=False)
        issue(1, 1, wait_scatter=False)

        def pair(g, carry):
            consume(0)

            @pl.when(2 * g + 2 < nblk)
            def _():
                issue(2 * g + 2, 0, wait_scatter=True)

            consume(1)

            @pl.when(2 * g + 3 < nblk)
            def _():
                issue(2 * g + 3, 1, wait_scatter=True)

            return carry

        lax.fori_loop(0, nblk // 2, pair, 0)

        @pl.when(nblk % 2 == 1)
        def _():
            consume(0)

        for s in range(2):
            db, srb, adb, asb, eb, ab, bb, se, sa, sb, ss = slots[s]
            pltpu.make_async_copy(eb, shared.at[db.at[0]], ss).wait()

    zero_table()
    plsc.subcore_barrier()
    for r in range(NCHUNK // 2):
        chunk = core * (NCHUNK // 2) + r
        run_round(chunk)
        plsc.subcore_barrier()
        row0 = sid * ROWS_PER_TILE
        pltpu.sync_copy(shared.at[pl.ds(row0, ROWS_PER_TILE)],
                        out_hbm.at[pl.ds(chunk * N_PAD + row0, ROWS_PER_TILE)])
        if r < NCHUNK // 2 - 1:
            zero_table()
        plsc.subcore_barrier()


def _sc_aggregate(dst2, src2, eaw, xiw, xjw, zeros):
    mesh = plsc.VectorSubcoreMesh(core_axis_name="c", subcore_axis_name="s")
    f = pl.kernel(
        _sc_body,
        mesh=mesh,
        out_type=jax.ShapeDtypeStruct((NCHUNK * N_PAD, C), jnp.float32),
        scratch_types=(
            [pltpu.VMEM_SHARED((N_PAD, C), jnp.float32)]
            + [pltpu.VMEM((1, 128), jnp.int32) for _ in range(8)]
            + [pltpu.VMEM((BLK, C), jnp.float32) for _ in range(6)]
            + [pltpu.SemaphoreType.DMA for _ in range(8)]
        ),
        compiler_params=pltpu.CompilerParams(use_tc_tiling_on_sc=False),
    )
    return f(dst2, src2, eaw, xiw, xjw, zeros)


def _mlp_body(aggr_ref, x_ref, w1_ref, b1_ref, w2_ref, b2_ref, out_ref):
    cat = jnp.concatenate([aggr_ref[c] for c in range(NCHUNK)], axis=1)
    out_node = cat[:, :D] + x_ref[...]
    h = jax.nn.relu(jnp.dot(out_node, w1_ref[...],
                            preferred_element_type=jnp.float32,
                            precision=lax.Precision.HIGHEST) + b1_ref[...])
    out_ref[...] = jnp.dot(h, w2_ref[...],
                           preferred_element_type=jnp.float32,
                           precision=lax.Precision.HIGHEST) + b2_ref[...]


def _mlp(aggr4, x, w1t, b1, w2t, b2):
    bn = 2000
    return pl.pallas_call(
        _mlp_body,
        grid=(N // bn,),
        in_specs=[
            pl.BlockSpec((NCHUNK, bn, C), lambda i: (0, i, 0)),
            pl.BlockSpec((bn, D), lambda i: (i, 0)),
            pl.BlockSpec((D, D), lambda i: (0, 0)),
            pl.BlockSpec((1, D), lambda i: (0, 0)),
            pl.BlockSpec((D, D), lambda i: (0, 0)),
            pl.BlockSpec((1, D), lambda i: (0, 0)),
        ],
        out_specs=pl.BlockSpec((bn, D), lambda i: (i, 0)),
        out_shape=jax.ShapeDtypeStruct((N, D), jnp.float32),
    )(aggr4, x, w1t, b1, w2t, b2)


def kernel(x, edge_index, edge_attr, W_lin, b_lin, W1, b1, W2, b2):
    wi = W_lin[:, :D].T
    we = W_lin[:, D:2 * D].T
    wj = W_lin[:, 2 * D:].T
    pad = ((0, 0), (0, DP - D))
    wi_pad = jnp.pad(wi, pad)
    we_pad = jnp.pad(we, pad)
    wj_pad = jnp.pad(wj, pad)
    b_pad = jnp.pad(b_lin, (0, DP - D)).reshape(1, DP)

    eaw = _edge_mm(edge_attr, we_pad, b_pad)
    xiw4, xjw4 = _node_mm(x, wi_pad, wj_pad)
    xiw = xiw4.reshape(NCHUNK * N, C)
    xjw = xjw4.reshape(NCHUNK * N, C)

    dst2 = edge_index[1].astype(jnp.int32).reshape(NBLK, 128)
    src2 = edge_index[0].astype(jnp.int32).reshape(NBLK, 128)
    zeros = jnp.zeros((ROWS_PER_TILE, C), jnp.float32)

    aggr = _sc_aggregate(dst2, src2, eaw, xiw, xjw, zeros)
    aggr4 = aggr.reshape(NCHUNK, N_PAD, C)

    return _mlp(aggr4, x, W1.T, b1.reshape(1, D), W2.T, b2.reshape(1, D))


# speedup vs baseline: 1.3058x; 1.3058x over previous
import functools

import jax
import jax.numpy as jnp
from jax import lax
from jax.experimental import pallas as pl
from jax.experimental.pallas import tpu as pltpu
from jax.experimental.pallas import tpu_sc as plsc

N = 50000
E = 800000
D = 100
DP = 128
C = 32
NCHUNK = DP // C
BLK = 128
NBLK = E // BLK
N_PAD = 50048
ROWS_PER_TILE = N_PAD // 16


def _edge_mm_body(ea_ref, w_ref, b_ref, out_ref):
    out_ref[...] = jnp.dot(ea_ref[...], w_ref[...],
                           preferred_element_type=jnp.float32,
                           precision=lax.Precision.HIGHEST) + b_ref[...]


def _edge_mm(edge_attr, w_pad, b_pad):
    bk = 2000
    return pl.pallas_call(
        _edge_mm_body,
        grid=(E // bk,),
        in_specs=[
            pl.BlockSpec((bk, D), lambda i: (i, 0)),
            pl.BlockSpec((D, DP), lambda i: (0, 0)),
            pl.BlockSpec((1, DP), lambda i: (0, 0)),
        ],
        out_specs=pl.BlockSpec((bk, DP), lambda i: (i, 0)),
        out_shape=jax.ShapeDtypeStruct((E, DP), jnp.float32),
    )(edge_attr, w_pad, b_pad)


def _node_mm_body(x_ref, wi_ref, wj_ref, oi_ref, oj_ref):
    yi = jnp.dot(x_ref[...], wi_ref[...],
                 preferred_element_type=jnp.float32,
                 precision=lax.Precision.HIGHEST)
    yj = jnp.dot(x_ref[...], wj_ref[...],
                 preferred_element_type=jnp.float32,
                 precision=lax.Precision.HIGHEST)
    for c in range(NCHUNK):
        oi_ref[c] = yi[:, c * C:(c + 1) * C]
        oj_ref[c] = yj[:, c * C:(c + 1) * C]


def _node_mm(x, wi_pad, wj_pad):
    bn = 2000
    return pl.pallas_call(
        _node_mm_body,
        grid=(N // bn,),
        in_specs=[
            pl.BlockSpec((bn, D), lambda i: (i, 0)),
            pl.BlockSpec((D, DP), lambda i: (0, 0)),
            pl.BlockSpec((D, DP), lambda i: (0, 0)),
        ],
        out_specs=[
            pl.BlockSpec((NCHUNK, bn, C), lambda i: (0, i, 0)),
            pl.BlockSpec((NCHUNK, bn, C), lambda i: (0, i, 0)),
        ],
        out_shape=[
            jax.ShapeDtypeStruct((NCHUNK, N, C), jnp.float32),
            jax.ShapeDtypeStruct((NCHUNK, N, C), jnp.float32),
        ],
    )(x, wi_pad, wj_pad)


def _sc_body(pairs_hbm, eaw_hbm, xiw_hbm, xjw_hbm, zeros_hbm,
             out_hbm, shared, dring,
             ad0, ad1, as0, as1, e0, e1, a0, a1, b0, b1,
             si0, si1, se0, se1, sa0, sa1, sb0, sb1, ss0, ss1):
    core = lax.axis_index("c")
    sid = lax.axis_index("s")
    slots = ((ad0, as0, e0, a0, b0, si0, se0, sa0, sb0, ss0),
             (ad1, as1, e1, a1, b1, si1, se1, sa1, sb1, ss1))

    def zero_table():
        pltpu.sync_copy(zeros_hbm,
                        shared.at[pl.ds(sid * ROWS_PER_TILE, ROWS_PER_TILE)])

    def run_round(chunk):
        ccol = pl.ds(chunk * C, C)
        cn = chunk * N

        def prefetch(j, si):
            blk = sid + 16 * j
            pltpu.async_copy(pairs_hbm.at[pl.ds(2 * blk, 2)],
                             dring.at[jnp.remainder(j, 4)], si)

        def issue(j, s, wait_scatter):
            adb, asb, eb, ab, bb, si, se, sa, sb, ss = slots[s]
            jm = jnp.remainder(j, 4)
            if wait_scatter:
                pltpu.make_async_copy(eb, shared.at[dring.at[jm, 0]],
                                      ss).wait()
            blk = sid + 16 * j
            pltpu.make_async_copy(pairs_hbm.at[pl.ds(0, 2)],
                                  dring.at[jm], si).wait()
            for v in range(8):
                sl = pl.ds(v * 16, 16)
                adb[0, sl] = dring[jm, 0, sl] + cn
                asb[0, sl] = dring[jm, 1, sl] + cn
            pltpu.async_copy(eaw_hbm.at[pl.ds(blk * BLK, BLK), ccol], eb, se)
            pltpu.async_copy(xiw_hbm.at[adb.at[0]], ab, sa)
            pltpu.async_copy(xjw_hbm.at[asb.at[0]], bb, sb)
            return jm

        def consume(s, jm):
            adb, asb, eb, ab, bb, si, se, sa, sb, ss = slots[s]
            pltpu.make_async_copy(eaw_hbm.at[pl.ds(0, BLK), ccol], eb,
                                  se).wait()
            pltpu.make_async_copy(xiw_hbm.at[adb.at[0]], ab, sa).wait()
            pltpu.make_async_copy(xjw_hbm.at[asb.at[0]], bb, sb).wait()

            def relu_row(r, acc):
                for h in range(2):
                    sl = pl.ds(h * 16, 16)
                    m = eb[r, sl] + ab[r, sl] + bb[r, sl]
                    eb[r, sl] = jnp.maximum(m, 0.0)
                return acc

            lax.fori_loop(0, BLK, relu_row, 0)
            pltpu.async_copy(eb, shared.at[dring.at[jm, 0]], ss, add=True)

        nblk = (NBLK // 16) + jnp.where(sid < (NBLK % 16), 1, 0)
        prefetch(0, si0)
        prefetch(1, si1)
        issue(0, 0, wait_scatter=False)
        prefetch(2, si0)
        issue(1, 1, wait_scatter=False)
        prefetch(3, si1)

        def pair(g, carry):
            consume(0, jnp.remainder(2 * g, 4))

            @pl.when(2 * g + 2 < nblk)
            def _():
                issue(2 * g + 2, 0, wait_scatter=True)

                @pl.when(2 * g + 4 < nblk)
                def _():
                    prefetch(2 * g + 4, si0)

            consume(1, jnp.remainder(2 * g + 1, 4))

            @pl.when(2 * g + 3 < nblk)
            def _():
                issue(2 * g + 3, 1, wait_scatter=True)

                @pl.when(2 * g + 5 < nblk)
                def _():
                    prefetch(2 * g + 5, si1)

            return carry

        lax.fori_loop(0, nblk // 2, pair, 0)

        @pl.when(nblk % 2 == 1)
        def _():
            consume(0, jnp.remainder(nblk - 1, 4))

        for s in range(2):
            adb, asb, eb, ab, bb, si, se, sa, sb, ss = slots[s]
            pltpu.make_async_copy(eb, shared.at[dring.at[0, 0]], ss).wait()

    zero_table()
    plsc.subcore_barrier()
    for r in range(NCHUNK // 2):
        chunk = core * (NCHUNK // 2) + r
        run_round(chunk)
        plsc.subcore_barrier()
        row0 = sid * ROWS_PER_TILE
        pltpu.sync_copy(shared.at[pl.ds(row0, ROWS_PER_TILE)],
                        out_hbm.at[pl.ds(chunk * N_PAD + row0, ROWS_PER_TILE)])
        if r < NCHUNK // 2 - 1:
            zero_table()
        plsc.subcore_barrier()


def _sc_aggregate(pairs2, eaw, xiw, xjw, zeros):
    mesh = plsc.VectorSubcoreMesh(core_axis_name="c", subcore_axis_name="s")
    f = pl.kernel(
        _sc_body,
        mesh=mesh,
        out_type=jax.ShapeDtypeStruct((NCHUNK * N_PAD, C), jnp.float32),
        scratch_types=(
            [pltpu.VMEM_SHARED((N_PAD, C), jnp.float32),
             pltpu.VMEM((4, 2, 128), jnp.int32)]
            + [pltpu.VMEM((1, 128), jnp.int32) for _ in range(4)]
            + [pltpu.VMEM((BLK, C), jnp.float32) for _ in range(6)]
            + [pltpu.SemaphoreType.DMA for _ in range(10)]
        ),
        compiler_params=pltpu.CompilerParams(use_tc_tiling_on_sc=False),
    )
    return f(pairs2, eaw, xiw, xjw, zeros)


def _mlp_body(aggr_ref, x_ref, w1_ref, b1_ref, w2_ref, b2_ref, out_ref):
    cat = jnp.concatenate([aggr_ref[c] for c in range(NCHUNK)], axis=1)
    out_node = cat[:, :D] + x_ref[...]
    h = jax.nn.relu(jnp.dot(out_node, w1_ref[...],
                            preferred_element_type=jnp.float32,
                            precision=lax.Precision.HIGHEST) + b1_ref[...])
    out_ref[...] = jnp.dot(h, w2_ref[...],
                           preferred_element_type=jnp.float32,
                           precision=lax.Precision.HIGHEST) + b2_ref[...]


def _mlp(aggr4, x, w1t, b1, w2t, b2):
    bn = 2000
    return pl.pallas_call(
        _mlp_body,
        grid=(N // bn,),
        in_specs=[
            pl.BlockSpec((NCHUNK, bn, C), lambda i: (0, i, 0)),
            pl.BlockSpec((bn, D), lambda i: (i, 0)),
            pl.BlockSpec((D, D), lambda i: (0, 0)),
            pl.BlockSpec((1, D), lambda i: (0, 0)),
            pl.BlockSpec((D, D), lambda i: (0, 0)),
            pl.BlockSpec((1, D), lambda i: (0, 0)),
        ],
        out_specs=pl.BlockSpec((bn, D), lambda i: (i, 0)),
        out_shape=jax.ShapeDtypeStruct((N, D), jnp.float32),
    )(aggr4, x, w1t, b1, w2t, b2)


def kernel(x, edge_index, edge_attr, W_lin, b_lin, W1, b1, W2, b2):
    wi = W_lin[:, :D].T
    we = W_lin[:, D:2 * D].T
    wj = W_lin[:, 2 * D:].T
    pad = ((0, 0), (0, DP - D))
    wi_pad = jnp.pad(wi, pad)
    we_pad = jnp.pad(we, pad)
    wj_pad = jnp.pad(wj, pad)
    b_pad = jnp.pad(b_lin, (0, DP - D)).reshape(1, DP)

    eaw = _edge_mm(edge_attr, we_pad, b_pad)
    xiw4, xjw4 = _node_mm(x, wi_pad, wj_pad)
    xiw = xiw4.reshape(NCHUNK * N, C)
    xjw = xjw4.reshape(NCHUNK * N, C)

    dst2 = edge_index[1].astype(jnp.int32).reshape(NBLK, 128)
    src2 = edge_index[0].astype(jnp.int32).reshape(NBLK, 128)
    pairs2 = jnp.stack([dst2, src2], axis=1).reshape(2 * NBLK, 128)
    zeros = jnp.zeros((ROWS_PER_TILE, C), jnp.float32)

    aggr = _sc_aggregate(pairs2, eaw, xiw, xjw, zeros)
    aggr4 = aggr.reshape(NCHUNK, N_PAD, C)

    return _mlp(aggr4, x, W1.T, b1.reshape(1, D), W2.T, b2.reshape(1, D))


# speedup vs baseline: 1.4210x; 1.0882x over previous
import functools

import jax
import jax.numpy as jnp
from jax import lax
from jax.experimental import pallas as pl
from jax.experimental.pallas import tpu as pltpu
from jax.experimental.pallas import tpu_sc as plsc

N = 50000
E = 800000
D = 100
DP = 128
C = 32
NCHUNK = DP // C
BLK = 128
NBLK = E // BLK
N_PAD = 50048
ROWS_PER_TILE = N_PAD // 16


def _edge_mm_body(ea_ref, w_ref, b_ref, out_ref):
    out_ref[...] = jnp.dot(ea_ref[...], w_ref[...],
                           preferred_element_type=jnp.float32,
                           precision=lax.Precision.HIGHEST) + b_ref[...]


def _edge_mm(edge_attr, w_pad, b_pad):
    bk = 8000
    return pl.pallas_call(
        _edge_mm_body,
        grid=(E // bk,),
        in_specs=[
            pl.BlockSpec((bk, D), lambda i: (i, 0)),
            pl.BlockSpec((D, DP), lambda i: (0, 0)),
            pl.BlockSpec((1, DP), lambda i: (0, 0)),
        ],
        out_specs=pl.BlockSpec((bk, DP), lambda i: (i, 0)),
        out_shape=jax.ShapeDtypeStruct((E, DP), jnp.float32),
    )(edge_attr, w_pad, b_pad)


def _node_mm_body(x_ref, wi_ref, wj_ref, oi_ref, oj_ref):
    yi = jnp.dot(x_ref[...], wi_ref[...],
                 preferred_element_type=jnp.float32,
                 precision=lax.Precision.HIGHEST)
    yj = jnp.dot(x_ref[...], wj_ref[...],
                 preferred_element_type=jnp.float32,
                 precision=lax.Precision.HIGHEST)
    for c in range(NCHUNK):
        oi_ref[c] = yi[:, c * C:(c + 1) * C]
        oj_ref[c] = yj[:, c * C:(c + 1) * C]


def _node_mm(x, wi_pad, wj_pad):
    bn = 2000
    return pl.pallas_call(
        _node_mm_body,
        grid=(N // bn,),
        in_specs=[
            pl.BlockSpec((bn, D), lambda i: (i, 0)),
            pl.BlockSpec((D, DP), lambda i: (0, 0)),
            pl.BlockSpec((D, DP), lambda i: (0, 0)),
        ],
        out_specs=[
            pl.BlockSpec((NCHUNK, bn, C), lambda i: (0, i, 0)),
            pl.BlockSpec((NCHUNK, bn, C), lambda i: (0, i, 0)),
        ],
        out_shape=[
            jax.ShapeDtypeStruct((NCHUNK, N, C), jnp.float32),
            jax.ShapeDtypeStruct((NCHUNK, N, C), jnp.float32),
        ],
    )(x, wi_pad, wj_pad)


def _sc_body(pairs_hbm, eaw_hbm, xiw_hbm, xjw_hbm, zeros_hbm,
             out_hbm, shared, dring,
             ad0, ad1, as0, as1, e0, e1, a0, a1, b0, b1,
             si0, si1, se0, se1, sa0, sa1, sb0, sb1, ss0, ss1):
    core = lax.axis_index("c")
    sid = lax.axis_index("s")
    slots = ((ad0, as0, e0, a0, b0, si0, se0, sa0, sb0, ss0),
             (ad1, as1, e1, a1, b1, si1, se1, sa1, sb1, ss1))

    def zero_table():
        pltpu.sync_copy(zeros_hbm,
                        shared.at[pl.ds(sid * ROWS_PER_TILE, ROWS_PER_TILE)])

    def run_round(chunk):
        ccol = pl.ds(chunk * C, C)
        cn = chunk * N

        def prefetch(j, si):
            blk = sid + 16 * j
            pltpu.async_copy(pairs_hbm.at[pl.ds(2 * blk, 2)],
                             dring.at[jnp.remainder(j, 4)], si)

        def issue(j, s, wait_scatter):
            adb, asb, eb, ab, bb, si, se, sa, sb, ss = slots[s]
            jm = jnp.remainder(j, 4)
            if wait_scatter:
                pltpu.make_async_copy(eb, shared.at[dring.at[jm, 0]],
                                      ss).wait()
            blk = sid + 16 * j
            pltpu.make_async_copy(pairs_hbm.at[pl.ds(0, 2)],
                                  dring.at[jm], si).wait()
            for v in range(8):
                sl = pl.ds(v * 16, 16)
                adb[0, sl] = dring[jm, 0, sl] + cn
                asb[0, sl] = dring[jm, 1, sl] + cn
            pltpu.async_copy(eaw_hbm.at[pl.ds(blk * BLK, BLK), ccol], eb, se)
            pltpu.async_copy(xiw_hbm.at[adb.at[0]], ab, sa)
            pltpu.async_copy(xjw_hbm.at[asb.at[0]], bb, sb)
            return jm

        def consume(s, jm):
            adb, asb, eb, ab, bb, si, se, sa, sb, ss = slots[s]
            pltpu.make_async_copy(eaw_hbm.at[pl.ds(0, BLK), ccol], eb,
                                  se).wait()
            pltpu.make_async_copy(xiw_hbm.at[adb.at[0]], ab, sa).wait()
            pltpu.make_async_copy(xjw_hbm.at[asb.at[0]], bb, sb).wait()

            def relu_row(r, acc):
                for h in range(2):
                    sl = pl.ds(h * 16, 16)
                    m = eb[r, sl] + ab[r, sl] + bb[r, sl]
                    eb[r, sl] = jnp.maximum(m, 0.0)
                return acc

            lax.fori_loop(0, BLK, relu_row, 0)
            pltpu.async_copy(eb, shared.at[dring.at[jm, 0]], ss, add=True)

        nblk = (NBLK // 16) + jnp.where(sid < (NBLK % 16), 1, 0)
        prefetch(0, si0)
        prefetch(1, si1)
        issue(0, 0, wait_scatter=False)
        prefetch(2, si0)
        issue(1, 1, wait_scatter=False)
        prefetch(3, si1)

        def pair(g, carry):
            consume(0, jnp.remainder(2 * g, 4))

            @pl.when(2 * g + 2 < nblk)
            def _():
                issue(2 * g + 2, 0, wait_scatter=True)

                @pl.when(2 * g + 4 < nblk)
                def _():
                    prefetch(2 * g + 4, si0)

            consume(1, jnp.remainder(2 * g + 1, 4))

            @pl.when(2 * g + 3 < nblk)
            def _():
                issue(2 * g + 3, 1, wait_scatter=True)

                @pl.when(2 * g + 5 < nblk)
                def _():
                    prefetch(2 * g + 5, si1)

            return carry

        lax.fori_loop(0, nblk // 2, pair, 0)

        @pl.when(nblk % 2 == 1)
        def _():
            consume(0, jnp.remainder(nblk - 1, 4))

        for s in range(2):
            adb, asb, eb, ab, bb, si, se, sa, sb, ss = slots[s]
            pltpu.make_async_copy(eb, shared.at[dring.at[0, 0]], ss).wait()

    zero_table()
    plsc.subcore_barrier()
    for r in range(NCHUNK // 2):
        chunk = core * (NCHUNK // 2) + r
        run_round(chunk)
        plsc.subcore_barrier()
        row0 = sid * ROWS_PER_TILE
        pltpu.sync_copy(shared.at[pl.ds(row0, ROWS_PER_TILE)],
                        out_hbm.at[pl.ds(chunk * N_PAD + row0, ROWS_PER_TILE)])
        if r < NCHUNK // 2 - 1:
            zero_table()
        plsc.subcore_barrier()


def _sc_aggregate(pairs2, eaw, xiw, xjw, zeros):
    mesh = plsc.VectorSubcoreMesh(core_axis_name="c", subcore_axis_name="s")
    f = pl.kernel(
        _sc_body,
        mesh=mesh,
        out_type=jax.ShapeDtypeStruct((NCHUNK * N_PAD, C), jnp.float32),
        scratch_types=(
            [pltpu.VMEM_SHARED((N_PAD, C), jnp.float32),
             pltpu.VMEM((4, 2, 128), jnp.int32)]
            + [pltpu.VMEM((1, 128), jnp.int32) for _ in range(4)]
            + [pltpu.VMEM((BLK, C), jnp.float32) for _ in range(6)]
            + [pltpu.SemaphoreType.DMA for _ in range(10)]
        ),
        compiler_params=pltpu.CompilerParams(use_tc_tiling_on_sc=False),
    )
    return f(pairs2, eaw, xiw, xjw, zeros)


def _mlp_body(aggr_ref, x_ref, w1_ref, b1_ref, w2_ref, b2_ref, out_ref):
    cat = jnp.concatenate([aggr_ref[c] for c in range(NCHUNK)], axis=1)
    out_node = cat[:, :D] + x_ref[...]
    h = jax.nn.relu(jnp.dot(out_node, w1_ref[...],
                            preferred_element_type=jnp.float32,
                            precision=lax.Precision.HIGHEST) + b1_ref[...])
    out_ref[...] = jnp.dot(h, w2_ref[...],
                           preferred_element_type=jnp.float32,
                           precision=lax.Precision.HIGHEST) + b2_ref[...]


def _mlp(aggr4, x, w1t, b1, w2t, b2):
    bn = 2000
    return pl.pallas_call(
        _mlp_body,
        grid=(N // bn,),
        in_specs=[
            pl.BlockSpec((NCHUNK, bn, C), lambda i: (0, i, 0)),
            pl.BlockSpec((bn, D), lambda i: (i, 0)),
            pl.BlockSpec((D, D), lambda i: (0, 0)),
            pl.BlockSpec((1, D), lambda i: (0, 0)),
            pl.BlockSpec((D, D), lambda i: (0, 0)),
            pl.BlockSpec((1, D), lambda i: (0, 0)),
        ],
        out_specs=pl.BlockSpec((bn, D), lambda i: (i, 0)),
        out_shape=jax.ShapeDtypeStruct((N, D), jnp.float32),
    )(aggr4, x, w1t, b1, w2t, b2)


def kernel(x, edge_index, edge_attr, W_lin, b_lin, W1, b1, W2, b2):
    wi = W_lin[:, :D].T
    we = W_lin[:, D:2 * D].T
    wj = W_lin[:, 2 * D:].T
    pad = ((0, 0), (0, DP - D))
    wi_pad = jnp.pad(wi, pad)
    we_pad = jnp.pad(we, pad)
    wj_pad = jnp.pad(wj, pad)
    b_pad = jnp.pad(b_lin, (0, DP - D)).reshape(1, DP)

    eaw = _edge_mm(edge_attr, we_pad, b_pad)
    xiw4, xjw4 = _node_mm(x, wi_pad, wj_pad)
    xiw = xiw4.reshape(NCHUNK * N, C)
    xjw = xjw4.reshape(NCHUNK * N, C)

    dst2 = edge_index[1].astype(jnp.int32).reshape(NBLK, 128)
    src2 = edge_index[0].astype(jnp.int32).reshape(NBLK, 128)
    pairs2 = jnp.stack([dst2, src2], axis=1).reshape(2 * NBLK, 128)
    zeros = jnp.zeros((ROWS_PER_TILE, C), jnp.float32)

    aggr = _sc_aggregate(pairs2, eaw, xiw, xjw, zeros)
    aggr4 = aggr.reshape(NCHUNK, N_PAD, C)

    return _mlp(aggr4, x, W1.T, b1.reshape(1, D), W2.T, b2.reshape(1, D))
